# baseline (speedup 1.0000x reference)
import functools

import jax
import jax.numpy as jnp
from jax import lax
from jax.experimental import pallas as pl
from jax.experimental.pallas import tpu as pltpu
from jax.experimental.pallas import tpu_sc as plsc

NC = 2
NS = 16
NW = NC * NS

CH = 128
NB = 8


def _transpose_table(table_t, tail_lin, v, d):
    mesh = plsc.VectorSubcoreMesh(core_axis_name="c", subcore_axis_name="s")
    ntiles = v // 128
    tail = v - ntiles * 128

    @functools.partial(
        pl.kernel,
        mesh=mesh,
        out_type=jax.ShapeDtypeStruct((v * d,), jnp.float32),
        scratch_types=[
            pltpu.VMEM((d, 128), jnp.float32),
            pltpu.VMEM((128 * d,), jnp.float32),
        ],
        compiler_params=pltpu.CompilerParams(
            use_tc_tiling_on_sc=True, needs_layout_passes=False),
    )
    def k(tab_hbm, tail_hbm, out_hbm, slab_v, lin_v):
        wid = lax.axis_index("s") * NC + lax.axis_index("c")
        q, r = divmod(ntiles, NW)
        start = wid * q + jnp.minimum(wid, r)
        cnt = q + (wid < r).astype(jnp.int32)

        @pl.loop(start, start + cnt)
        def _(t):
            pltpu.sync_copy(tab_hbm.at[:, pl.ds(t * 128, 128)], slab_v)

            @pl.loop(0, 128, unroll=4)
            def _(c):
                for dg in range(d // 16):
                    vals = plsc.load_gather(
                        slab_v,
                        [lax.iota(jnp.int32, 16) + 16 * dg,
                         jnp.full((16,), c, jnp.int32)])
                    lin_v[pl.ds(c * d + 16 * dg, 16)] = vals

            pltpu.sync_copy(lin_v, out_hbm.at[pl.ds(t * 128 * d, 128 * d)])

        if tail:
            @pl.when(wid == NW - 1)
            def _():
                pltpu.sync_copy(tail_hbm, lin_v.at[pl.ds(0, tail * d)])
                pltpu.sync_copy(
                    lin_v.at[pl.ds(0, tail * d)],
                    out_hbm.at[pl.ds(ntiles * 128 * d, tail * d)])

    return k(table_t, tail_lin)


@functools.partial(jax.jit, static_argnums=(2, 3, 4))
def _emb_lookup(xr, table, total, d, nch):
    mesh = plsc.VectorSubcoreMesh(core_axis_name="c", subcore_axis_name="s")
    b_per_w = nch * CH

    @functools.partial(
        pl.kernel,
        mesh=mesh,
        out_type=jax.ShapeDtypeStruct((total, d), jnp.float32),
        scratch_types=[
            pltpu.VMEM((nch, CH), jnp.int32),
            pltpu.VMEM((NB, CH, d), jnp.float32),
            pltpu.SemaphoreType.DMA,
            pltpu.SemaphoreType.DMA,
        ],
        compiler_params=pltpu.CompilerParams(use_tc_tiling_on_sc=False),
    )
    def k(x_hbm, tab_hbm, out_hbm, idx_v, rows_v, gsem, ssem):
        wid = lax.axis_index("s") * NC + lax.axis_index("c")
        base = wid * b_per_w
        pltpu.sync_copy(x_hbm.at[wid], idx_v)

        def gather(c, b):
            pltpu.async_copy(tab_hbm.at[idx_v.at[c]], rows_v.at[b], gsem)

        def wait_gather(b):
            pltpu.make_async_copy(
                tab_hbm.at[idx_v.at[0]], rows_v.at[b], gsem).wait()

        def wait_scatter():
            pltpu.make_async_copy(
                rows_v.at[0], out_hbm.at[pl.ds(base, CH)], ssem).wait()

        for b in range(NB):
            gather(b, b)

        @pl.loop(0, nch // NB)
        def _(p):
            for b in range(NB):
                s = p * NB + b
                wait_gather(b)
                pltpu.async_copy(
                    rows_v.at[b], out_hbm.at[pl.ds(base + s * CH, CH)], ssem)
                @pl.when(jnp.logical_and(s >= 2, s < nch - NB + 2))
                def _():
                    wait_scatter()
                    gather(s + NB - 2, (b - 2) % NB)

        for _ in range(NB):
            wait_scatter()

    v = table.shape[0]
    tail_lin = table[(v // 128) * 128:, :].reshape(-1)
    table_lin = _transpose_table(table.T, tail_lin, v, d).reshape(v, d)
    return k(xr, table_lin)


def kernel(x, embeddings):
    b, h = x.shape
    _, d = embeddings.shape
    total = b * h
    b_per_w = total // NW
    nch = b_per_w // CH
    xr = x.reshape(NW, nch, CH).astype(jnp.int32)
    out = _emb_lookup(xr, embeddings, total, d, nch)
    return out.reshape(b, h, d)
